# baseline (speedup 1.0000x reference)
import functools

import jax
import jax.numpy as jnp
from jax import lax
from jax.experimental import pallas as pl
from jax.experimental.pallas import tpu as pltpu
from jax.experimental.pallas import tpu_sc as plsc

BATCH = 4096
HIST = 200
EMBED_DIM = 64

_NC, _NS = 2, 16
_NW = _NC * _NS
_RPW = BATCH // _NW
_CB = 2
_NCHUNK = _RPW // _CB
_NB = 2
_NGROUP = _NCHUNK // _NB

_mesh = plsc.VectorSubcoreMesh(core_axis_name="c", subcore_axis_name="s")


@functools.partial(
    pl.kernel,
    mesh=_mesh,
    out_type=jax.ShapeDtypeStruct((BATCH, HIST, EMBED_DIM), jnp.float32),
    scratch_types=[
        pltpu.VMEM((_CB, HIST), jnp.int32),
        pltpu.VMEM((_CB, HIST), jnp.int32),
        pltpu.VMEM((_CB, HIST, EMBED_DIM), jnp.float32),
        pltpu.VMEM((_CB, HIST, EMBED_DIM), jnp.float32),
        pltpu.SemaphoreType.DMA,
        pltpu.SemaphoreType.DMA,
        pltpu.SemaphoreType.DMA,
        pltpu.SemaphoreType.DMA,
    ],
    compiler_params=pltpu.CompilerParams(use_tc_tiling_on_sc=False),
)
def _embed_lookup(tok_hbm, table_hbm, out_hbm, idx0, idx1, rows0, rows1,
                  gsem0, gsem1, osem0, osem1):
    idx_bufs = (idx0, idx1)
    row_bufs = (rows0, rows1)
    gsems = (gsem0, gsem1)
    osems = (osem0, osem1)

    wid = lax.axis_index("s") * _NC + lax.axis_index("c")
    base = wid * _RPW

    def start_gathers(b):
        for r in range(_CB):
            pltpu.async_copy(table_hbm.at[idx_bufs[b].at[r]],
                             row_bufs[b].at[r], gsems[b])

    def wait_gathers(b):
        for r in range(_CB):
            pltpu.make_async_copy(table_hbm.at[idx_bufs[b].at[r]],
                                  row_bufs[b].at[r], gsems[b]).wait()

    for b in range(_NB):
        off = base + b * _CB
        pltpu.sync_copy(tok_hbm.at[pl.ds(off, _CB)], idx_bufs[b])
        start_gathers(b)

    def group(i, carry):
        for b in range(_NB):
            off = base + (i * _NB + b) * _CB
            wait_gathers(b)
            pltpu.async_copy(row_bufs[b], out_hbm.at[pl.ds(off, _CB)],
                             osems[b])
        for b in range(_NB):
            off = base + (i * _NB + b) * _CB
            noff = base + ((i + 1) * _NB + b) * _CB
            more = i + 1 < _NGROUP

            @pl.when(more)
            def _():
                pltpu.sync_copy(tok_hbm.at[pl.ds(noff, _CB)], idx_bufs[b])

            pltpu.make_async_copy(
                row_bufs[b], out_hbm.at[pl.ds(off, _CB)], osems[b]).wait()

            @pl.when(more)
            def _():
                start_gathers(b)
        return carry

    lax.fori_loop(0, _NGROUP, group, 0)


def kernel(tokens, table):
    return _embed_lookup(tokens, table)


# speedup vs baseline: 1.0941x; 1.0941x over previous
import functools

import jax
import jax.numpy as jnp
from jax import lax
from jax.experimental import pallas as pl
from jax.experimental.pallas import tpu as pltpu
from jax.experimental.pallas import tpu_sc as plsc

BATCH = 4096
HIST = 200
EMBED_DIM = 64
PAD_DIM = 128

_NC, _NS = 2, 16
_NW = _NC * _NS
_RPW = BATCH // _NW
_NCHUNK = _RPW
_NB = 2
_NGROUP = _NCHUNK // _NB

_mesh = plsc.VectorSubcoreMesh(core_axis_name="c", subcore_axis_name="s")


@functools.partial(
    pl.kernel,
    mesh=_mesh,
    out_type=jax.ShapeDtypeStruct((BATCH, HIST, EMBED_DIM), jnp.float32),
    scratch_types=[
        pltpu.VMEM((HIST,), jnp.int32),
        pltpu.VMEM((HIST,), jnp.int32),
        pltpu.VMEM((HIST, PAD_DIM), jnp.float32),
        pltpu.VMEM((HIST, PAD_DIM), jnp.float32),
        pltpu.VMEM((HIST, EMBED_DIM), jnp.float32),
        pltpu.VMEM((HIST, EMBED_DIM), jnp.float32),
        pltpu.SemaphoreType.DMA,
        pltpu.SemaphoreType.DMA,
        pltpu.SemaphoreType.DMA,
        pltpu.SemaphoreType.DMA,
    ],
    compiler_params=pltpu.CompilerParams(use_tc_tiling_on_sc=True),
)
def _embed_lookup(tok_hbm, table_hbm, out_hbm, idx0, idx1, ga0, ga1, st0, st1,
                  gsem0, gsem1, osem0, osem1):
    idx_bufs = (idx0, idx1)
    gather_bufs = (ga0, ga1)
    stage_bufs = (st0, st1)
    gsems = (gsem0, gsem1)
    osems = (osem0, osem1)

    wid = lax.axis_index("s") * _NC + lax.axis_index("c")
    base = wid * _RPW

    def compact(b):
        def row(h, carry):
            for j in range(EMBED_DIM // 16):
                stage_bufs[b][h, pl.ds(j * 16, 16)] = (
                    gather_bufs[b][h, pl.ds(j * 16, 16)])
            return carry
        lax.fori_loop(0, HIST, row, 0)

    for b in range(_NB):
        off = (base + b) * HIST
        pltpu.sync_copy(tok_hbm.at[pl.ds(off, HIST)], idx_bufs[b])
        pltpu.async_copy(table_hbm.at[idx_bufs[b]], gather_bufs[b], gsems[b])

    def group(i, carry):
        for b in range(_NB):
            row_id = base + i * _NB + b
            pltpu.make_async_copy(table_hbm.at[idx_bufs[b]], gather_bufs[b],
                                  gsems[b]).wait()
            compact(b)
            pltpu.async_copy(stage_bufs[b], out_hbm.at[row_id], osems[b])
        for b in range(_NB):
            row_id = base + i * _NB + b
            nrow = base + (i + 1) * _NB + b
            more = i + 1 < _NGROUP

            @pl.when(more)
            def _():
                pltpu.sync_copy(tok_hbm.at[pl.ds(nrow * HIST, HIST)],
                                idx_bufs[b])

            pltpu.make_async_copy(stage_bufs[b], out_hbm.at[row_id],
                                  osems[b]).wait()

            @pl.when(more)
            def _():
                pltpu.async_copy(table_hbm.at[idx_bufs[b]], gather_bufs[b],
                                 gsems[b])
        return carry

    lax.fori_loop(0, _NGROUP, group, 0)


def kernel(tokens, table):
    tokens_flat = tokens.reshape(-1)
    table_pad = jnp.pad(table, ((0, 0), (0, PAD_DIM - EMBED_DIM)))
    return _embed_lookup(tokens_flat, table_pad)
